# baseline (speedup 1.0000x reference)
import jax
import jax.numpy as jnp
from jax.experimental import pallas as pl


_BS = 512


def _bcast_kernel(w_ref, o_ref, *, bsz, dim):
    w = w_ref[...]
    for b in range(bsz):
        o_ref[:, b * dim:(b + 1) * dim] = w


def kernel(input, weights):
    seq_len, bsz = input.shape
    init_size, dim = weights.shape
    bs = _BS if seq_len % _BS == 0 else seq_len
    grid = (seq_len // bs,)
    out = pl.pallas_call(
        lambda w_ref, o_ref: _bcast_kernel(w_ref, o_ref, bsz=bsz, dim=dim),
        grid=grid,
        in_specs=[pl.BlockSpec((bs, dim), lambda i: (i, 0))],
        out_specs=pl.BlockSpec((bs, bsz * dim), lambda i: (i, 0)),
        out_shape=jax.ShapeDtypeStruct((seq_len, bsz * dim), weights.dtype),
    )(weights[:seq_len])
    return out.reshape(seq_len, bsz, dim)
